# Initial kernel scaffold; baseline (speedup 1.0000x reference)
import functools

import jax
import jax.numpy as jnp
from jax import lax
from jax.experimental import pallas as pl
from jax.experimental.pallas import tpu as pltpu
from jax.experimental.pallas import tpu_sc as plsc

N = 10000
E = 320000
D = 128
G = 64

NC = 2
NS = 16
NW = NC * NS

CHUNK = 128
EP = 327680
CPW = EP // (NW * CHUNK)
NP = 10240
RPT = NP // NS

_f32 = jnp.float32


_sc_mesh = plsc.VectorSubcoreMesh(
    core_axis_name="c", subcore_axis_name="s", num_cores=NC, num_subcores=NS
)


@functools.partial(
    pl.kernel,
    out_type=(
        jax.ShapeDtypeStruct((NC, NP, D), _f32),
        jax.ShapeDtypeStruct((NC, NP, 16), _f32),
    ),
    mesh=_sc_mesh,
    scratch_types=[
        pltpu.VMEM((CPW, CHUNK), jnp.int32),
        pltpu.VMEM((CPW, CHUNK), jnp.int32),
        pltpu.VMEM((CHUNK, D), _f32),
        pltpu.VMEM((CHUNK, 16), _f32),
        pltpu.VMEM_SHARED((NP, D), _f32),
        pltpu.VMEM_SHARED((NP, 16), _f32),
    ],
)
def _sc_agg(h_hbm, src_hbm, dst_hbm, z128_hbm, zc_hbm, ones_hbm,
            outp_hbm, outc_hbm,
            src_v, dst_v, rows_v, ones_v, acc_sh, cacc_sh):
    c = lax.axis_index("c")
    s = lax.axis_index("s")
    wid = s * NC + c

    pltpu.sync_copy(src_hbm.at[pl.ds(wid * CPW, CPW)], src_v)
    pltpu.sync_copy(dst_hbm.at[pl.ds(wid * CPW, CPW)], dst_v)
    pltpu.sync_copy(ones_hbm, ones_v)

    for k in range(RPT // CHUNK):
        pltpu.sync_copy(z128_hbm, acc_sh.at[pl.ds(s * RPT + k * CHUNK, CHUNK)])
    pltpu.sync_copy(zc_hbm, cacc_sh.at[pl.ds(s * RPT, RPT)])
    plsc.subcore_barrier()

    def body(i, carry):
        pltpu.sync_copy(h_hbm.at[src_v.at[i]], rows_v)
        pltpu.sync_copy(rows_v, acc_sh.at[dst_v.at[i]], add=True)
        pltpu.sync_copy(ones_v, cacc_sh.at[dst_v.at[i]], add=True)
        return carry

    lax.fori_loop(0, CPW, body, 0)
    plsc.subcore_barrier()

    pltpu.sync_copy(acc_sh.at[pl.ds(s * RPT, RPT)],
                    outp_hbm.at[c, pl.ds(s * RPT, RPT)])
    pltpu.sync_copy(cacc_sh.at[pl.ds(s * RPT, RPT)],
                    outc_hbm.at[c, pl.ds(s * RPT, RPT)])


_BLK = 256
_NBLK = NP // _BLK


def _make_update(relu):
    def body(p_ref, inv_ref, h_ref, wl_ref, bl_ref, wr_ref, o_ref):
        mean = (p_ref[0] + p_ref[1]) * inv_ref[...]
        out = lax.dot_general(mean, wl_ref[...], (((1,), (1,)), ((), ())),
                              preferred_element_type=_f32)
        out += lax.dot_general(h_ref[...], wr_ref[...], (((1,), (1,)), ((), ())),
                               preferred_element_type=_f32)
        out += bl_ref[...]
        o_ref[...] = jnp.maximum(out, 0.0) if relu else out

    return pl.pallas_call(
        body,
        grid=(_NBLK,),
        in_specs=[
            pl.BlockSpec((2, _BLK, D), lambda i: (0, i, 0)),
            pl.BlockSpec((_BLK, 1), lambda i: (i, 0)),
            pl.BlockSpec((_BLK, D), lambda i: (i, 0)),
            pl.BlockSpec((D, D), lambda i: (0, 0)),
            pl.BlockSpec((1, D), lambda i: (0, 0)),
            pl.BlockSpec((D, D), lambda i: (0, 0)),
        ],
        out_specs=pl.BlockSpec((_BLK, D), lambda i: (i, 0)),
        out_shape=jax.ShapeDtypeStruct((NP, D), _f32),
    )


_update_relu = _make_update(True)


def _pool_body(batch_ref, p_ref, inv_ref, h2_ref, wl_ref, bl_ref, wr_ref,
               o_ref, a_acc, b_acc, c_acc):
    i = pl.program_id(0)

    @pl.when(i == 0)
    def _init():
        a_acc[...] = jnp.zeros((G, D), _f32)
        b_acc[...] = jnp.zeros((G, D), _f32)
        c_acc[...] = jnp.zeros((G, D), _f32)

    m3 = (p_ref[0] + p_ref[1]) * inv_ref[...]
    b = batch_ref[0]
    sel = (lax.broadcasted_iota(jnp.int32, (G, _BLK), 0) == b).astype(_f32)
    a_acc[...] += lax.dot_general(sel, m3, (((1,), (0,)), ((), ())),
                                  preferred_element_type=_f32)
    b_acc[...] += lax.dot_general(sel, h2_ref[...], (((1,), (0,)), ((), ())),
                                  preferred_element_type=_f32)
    c_acc[...] = c_acc[...] + jnp.sum(sel, axis=1, keepdims=True)

    @pl.when(i == _NBLK - 1)
    def _fin():
        cg = jnp.maximum(c_acc[...], 1.0)
        out = lax.dot_general(a_acc[...] / cg, wl_ref[...],
                              (((1,), (1,)), ((), ())),
                              preferred_element_type=_f32)
        out += lax.dot_general(b_acc[...] / cg, wr_ref[...],
                               (((1,), (1,)), ((), ())),
                               preferred_element_type=_f32)
        o_ref[...] = out + bl_ref[...]


_pool = pl.pallas_call(
    _pool_body,
    grid=(_NBLK,),
    in_specs=[
        pl.BlockSpec((1, 1, _BLK), lambda i: (i, 0, 0)),
        pl.BlockSpec((2, _BLK, D), lambda i: (0, i, 0)),
        pl.BlockSpec((_BLK, 1), lambda i: (i, 0)),
        pl.BlockSpec((_BLK, D), lambda i: (i, 0)),
        pl.BlockSpec((D, D), lambda i: (0, 0)),
        pl.BlockSpec((1, D), lambda i: (0, 0)),
        pl.BlockSpec((D, D), lambda i: (0, 0)),
    ],
    out_specs=pl.BlockSpec((G, D), lambda i: (0, 0)),
    out_shape=jax.ShapeDtypeStruct((G, D), _f32),
    scratch_shapes=[
        pltpu.VMEM((G, D), _f32),
        pltpu.VMEM((G, D), _f32),
        pltpu.VMEM((G, D), _f32),
    ],
)


def kernel(x, edge_index, batch, Wl1, bl1, Wr1, Wl2, bl2, Wr2, Wl3, bl3, Wr3):
    xp = jnp.zeros((NP, D), _f32).at[:N].set(x)
    src = jnp.concatenate(
        [edge_index[0], jnp.zeros((EP - E,), jnp.int32)]).reshape(-1, CHUNK)
    dst = jnp.concatenate(
        [edge_index[1], jnp.full((EP - E,), N, jnp.int32)]).reshape(-1, CHUNK)
    batchp = jnp.concatenate(
        [batch, jnp.full((NP - N,), 1 << 20, jnp.int32)]).reshape(_NBLK, 1, _BLK)
    z128 = jnp.zeros((CHUNK, D), _f32)
    zc = jnp.zeros((RPT, 16), _f32)
    ones16 = jnp.ones((CHUNK, 16), _f32)

    p1, c1 = _sc_agg(xp, src, dst, z128, zc, ones16)
    cnt = c1[0, :, 0] + c1[1, :, 0]
    inv = (1.0 / jnp.clip(cnt, 1.0, None)).reshape(NP, 1)

    h1 = _update_relu(p1, inv, xp, Wl1, bl1.reshape(1, D), Wr1)
    p2, _ = _sc_agg(h1, src, dst, z128, zc, ones16)
    h2 = _update_relu(p2, inv, h1, Wl2, bl2.reshape(1, D), Wr2)
    p3, _ = _sc_agg(h2, src, dst, z128, zc, ones16)
    return _pool(batchp, p3, inv, h2, Wl3, bl3.reshape(1, D), Wr3)


# speedup vs baseline: 3.0997x; 3.0997x over previous
import functools

import jax
import jax.numpy as jnp
from jax import lax
from jax.experimental import pallas as pl
from jax.experimental.pallas import tpu as pltpu
from jax.experimental.pallas import tpu_sc as plsc

N = 10000
E = 320000
D = 128
G = 64

NC = 2
NS = 16
NW = NC * NS

CHUNK = 128
EP = 327680
CPW = EP // (NW * CHUNK)
NP = 10240
RPT = NP // NS

_f32 = jnp.float32


_sc_mesh = plsc.VectorSubcoreMesh(
    core_axis_name="c", subcore_axis_name="s", num_cores=NC, num_subcores=NS
)


@functools.partial(
    pl.kernel,
    out_type=jax.ShapeDtypeStruct((NC, NP, D), _f32),
    mesh=_sc_mesh,
    scratch_types=[
        pltpu.VMEM((16, CHUNK), jnp.int32),
        pltpu.VMEM((16, CHUNK), jnp.int32),
        pltpu.VMEM((CHUNK, D), _f32),
        pltpu.VMEM_SHARED((NP, D), _f32),
    ],
)
def _sc_agg(h_hbm, src_hbm, dst_hbm, z128_hbm, outp_hbm,
            src_v, dst_v, rows_v, acc_sh):
    c = lax.axis_index("c")
    s = lax.axis_index("s")
    wid = s * NC + c

    pltpu.sync_copy(z128_hbm, rows_v)
    for k in range(RPT // CHUNK):
        pltpu.sync_copy(rows_v, acc_sh.at[pl.ds(s * RPT + k * CHUNK, CHUNK)])
    plsc.subcore_barrier()

    def body(i, carry):
        pltpu.sync_copy(h_hbm.at[src_v.at[i]], rows_v)
        pltpu.sync_copy(rows_v, acc_sh.at[dst_v.at[i]], add=True)
        return carry

    for blk in range(CPW // 16):
        pltpu.sync_copy(src_hbm.at[pl.ds(wid * CPW + blk * 16, 16)], src_v)
        pltpu.sync_copy(dst_hbm.at[pl.ds(wid * CPW + blk * 16, 16)], dst_v)
        lax.fori_loop(0, 16, body, 0)
    plsc.subcore_barrier()

    for k in range(RPT // CHUNK):
        r = s * RPT + k * CHUNK
        pltpu.sync_copy(acc_sh.at[pl.ds(r, CHUNK)], rows_v)
        pltpu.sync_copy(rows_v, outp_hbm.at[c, pl.ds(r, CHUNK)])


@functools.partial(
    pl.kernel,
    out_type=jax.ShapeDtypeStruct((NC, NP, D), _f32),
    mesh=_sc_mesh,
    scratch_types=[
        pltpu.VMEM((16, CHUNK), jnp.int32),
        pltpu.VMEM((CHUNK, D), _f32),
        pltpu.VMEM_SHARED((NP, D), _f32),
    ],
)
def _sc_count(dst_hbm, z128_hbm, ones_hbm, outc_hbm, dst_v, rows_v, acc_sh):
    c = lax.axis_index("c")
    s = lax.axis_index("s")
    wid = s * NC + c

    pltpu.sync_copy(z128_hbm, rows_v)
    for k in range(RPT // CHUNK):
        pltpu.sync_copy(rows_v, acc_sh.at[pl.ds(s * RPT + k * CHUNK, CHUNK)])
    plsc.subcore_barrier()

    pltpu.sync_copy(ones_hbm, rows_v)

    def body(i, carry):
        pltpu.sync_copy(rows_v, acc_sh.at[dst_v.at[i]], add=True)
        return carry

    for blk in range(CPW // 16):
        pltpu.sync_copy(dst_hbm.at[pl.ds(wid * CPW + blk * 16, 16)], dst_v)
        lax.fori_loop(0, 16, body, 0)
    plsc.subcore_barrier()

    for k in range(RPT // CHUNK):
        r = s * RPT + k * CHUNK
        pltpu.sync_copy(acc_sh.at[pl.ds(r, CHUNK)], rows_v)
        pltpu.sync_copy(rows_v, outc_hbm.at[c, pl.ds(r, CHUNK)])


_BLK = 256
_NBLK = NP // _BLK


def _make_update(relu):
    def body(p_ref, inv_ref, h_ref, wl_ref, bl_ref, wr_ref, o_ref):
        mean = (p_ref[0] + p_ref[1]) * inv_ref[...]
        out = lax.dot_general(mean, wl_ref[...], (((1,), (1,)), ((), ())),
                              preferred_element_type=_f32)
        out += lax.dot_general(h_ref[...], wr_ref[...], (((1,), (1,)), ((), ())),
                               preferred_element_type=_f32)
        out += bl_ref[...]
        o_ref[...] = jnp.maximum(out, 0.0) if relu else out

    return pl.pallas_call(
        body,
        grid=(_NBLK,),
        in_specs=[
            pl.BlockSpec((2, _BLK, D), lambda i: (0, i, 0)),
            pl.BlockSpec((_BLK, 1), lambda i: (i, 0)),
            pl.BlockSpec((_BLK, D), lambda i: (i, 0)),
            pl.BlockSpec((D, D), lambda i: (0, 0)),
            pl.BlockSpec((1, D), lambda i: (0, 0)),
            pl.BlockSpec((D, D), lambda i: (0, 0)),
        ],
        out_specs=pl.BlockSpec((_BLK, D), lambda i: (i, 0)),
        out_shape=jax.ShapeDtypeStruct((NP, D), _f32),
    )


_update_relu = _make_update(True)


def _pool_body(batch_ref, p_ref, inv_ref, h2_ref, wl_ref, bl_ref, wr_ref,
               o_ref, a_acc, b_acc, c_acc):
    i = pl.program_id(0)

    @pl.when(i == 0)
    def _init():
        a_acc[...] = jnp.zeros((G, D), _f32)
        b_acc[...] = jnp.zeros((G, D), _f32)
        c_acc[...] = jnp.zeros((G, D), _f32)

    m3 = (p_ref[0] + p_ref[1]) * inv_ref[...]
    b = batch_ref[0]
    sel = (lax.broadcasted_iota(jnp.int32, (G, _BLK), 0) == b).astype(_f32)
    a_acc[...] += lax.dot_general(sel, m3, (((1,), (0,)), ((), ())),
                                  preferred_element_type=_f32)
    b_acc[...] += lax.dot_general(sel, h2_ref[...], (((1,), (0,)), ((), ())),
                                  preferred_element_type=_f32)
    c_acc[...] = c_acc[...] + jnp.sum(sel, axis=1, keepdims=True)

    @pl.when(i == _NBLK - 1)
    def _fin():
        cg = jnp.maximum(c_acc[...], 1.0)
        out = lax.dot_general(a_acc[...] / cg, wl_ref[...],
                              (((1,), (1,)), ((), ())),
                              preferred_element_type=_f32)
        out += lax.dot_general(b_acc[...] / cg, wr_ref[...],
                               (((1,), (1,)), ((), ())),
                               preferred_element_type=_f32)
        o_ref[...] = out + bl_ref[...]


_pool = pl.pallas_call(
    _pool_body,
    grid=(_NBLK,),
    in_specs=[
        pl.BlockSpec((1, 1, _BLK), lambda i: (i, 0, 0)),
        pl.BlockSpec((2, _BLK, D), lambda i: (0, i, 0)),
        pl.BlockSpec((_BLK, 1), lambda i: (i, 0)),
        pl.BlockSpec((_BLK, D), lambda i: (i, 0)),
        pl.BlockSpec((D, D), lambda i: (0, 0)),
        pl.BlockSpec((1, D), lambda i: (0, 0)),
        pl.BlockSpec((D, D), lambda i: (0, 0)),
    ],
    out_specs=pl.BlockSpec((G, D), lambda i: (0, 0)),
    out_shape=jax.ShapeDtypeStruct((G, D), _f32),
    scratch_shapes=[
        pltpu.VMEM((G, D), _f32),
        pltpu.VMEM((G, D), _f32),
        pltpu.VMEM((G, D), _f32),
    ],
)


def kernel(x, edge_index, batch, Wl1, bl1, Wr1, Wl2, bl2, Wr2, Wl3, bl3, Wr3):
    xp = jnp.zeros((NP, D), _f32).at[:N].set(x)
    src = jnp.concatenate(
        [edge_index[0], jnp.zeros((EP - E,), jnp.int32)]).reshape(-1, CHUNK)
    dst = jnp.concatenate(
        [edge_index[1], jnp.full((EP - E,), N, jnp.int32)]).reshape(-1, CHUNK)
    batchp = jnp.concatenate(
        [batch, jnp.full((NP - N,), 1 << 20, jnp.int32)]).reshape(_NBLK, 1, _BLK)
    z128 = jnp.zeros((CHUNK, D), _f32)
    ones128 = jnp.ones((CHUNK, D), _f32)

    cp = _sc_count(dst, z128, ones128)
    inv = (1.0 / jnp.clip(cp[0, :, 0] + cp[1, :, 0], 1.0, None)).reshape(NP, 1)

    p1 = _sc_agg(xp, src, dst, z128)
    h1 = _update_relu(p1, inv, xp, Wl1, bl1.reshape(1, D), Wr1)
    p2 = _sc_agg(h1, src, dst, z128)
    h2 = _update_relu(p2, inv, h1, Wl2, bl2.reshape(1, D), Wr2)
    p3 = _sc_agg(h2, src, dst, z128)
    return _pool(batchp, p3, inv, h2, Wl3, bl3.reshape(1, D), Wr3)


# speedup vs baseline: 3.4389x; 1.1094x over previous
import functools

import jax
import jax.numpy as jnp
from jax import lax
from jax.experimental import pallas as pl
from jax.experimental.pallas import tpu as pltpu
from jax.experimental.pallas import tpu_sc as plsc

N = 10000
E = 320000
D = 128
G = 64

NC = 2
NS = 16
NW = NC * NS

CHUNK = 128
EP = 327680
CPW = EP // (NW * CHUNK)
NP = 10240
RPT = NP // NS

_f32 = jnp.float32


_sc_mesh = plsc.VectorSubcoreMesh(
    core_axis_name="c", subcore_axis_name="s", num_cores=NC, num_subcores=NS
)


@functools.partial(
    pl.kernel,
    out_type=jax.ShapeDtypeStruct((NC, NP, D), _f32),
    mesh=_sc_mesh,
    scratch_types=[
        pltpu.VMEM((16, CHUNK), jnp.int32),
        pltpu.VMEM((16, CHUNK), jnp.int32),
        pltpu.VMEM((2, CHUNK, D), _f32),
        pltpu.SemaphoreType.DMA,
        pltpu.SemaphoreType.DMA,
        pltpu.VMEM_SHARED((NP, D), _f32),
    ],
)
def _sc_agg(h_hbm, src_hbm, dst_hbm, z128_hbm, outp_hbm,
            src_v, dst_v, rows_v, sem0, sem1, acc_sh):
    c = lax.axis_index("c")
    s = lax.axis_index("s")
    wid = s * NC + c
    sems = (sem0, sem1)

    pltpu.sync_copy(z128_hbm, rows_v.at[0])
    for k in range(RPT // CHUNK):
        pltpu.sync_copy(rows_v.at[0],
                        acc_sh.at[pl.ds(s * RPT + k * CHUNK, CHUNK)])
    plsc.subcore_barrier()

    for blk in range(CPW // 16):
        pltpu.sync_copy(src_hbm.at[pl.ds(wid * CPW + blk * 16, 16)], src_v)
        pltpu.sync_copy(dst_hbm.at[pl.ds(wid * CPW + blk * 16, 16)], dst_v)
        pend = pltpu.async_copy(h_hbm.at[src_v.at[0]], rows_v.at[0], sem0)
        for k in range(16):
            nxt = None
            if k + 1 < 16:
                nxt = pltpu.async_copy(h_hbm.at[src_v.at[k + 1]],
                                       rows_v.at[(k + 1) % 2],
                                       sems[(k + 1) % 2])
            pend.wait()
            pltpu.sync_copy(rows_v.at[k % 2], acc_sh.at[dst_v.at[k]], add=True)
            pend = nxt
    plsc.subcore_barrier()

    for k in range(RPT // CHUNK):
        r = s * RPT + k * CHUNK
        pltpu.sync_copy(acc_sh.at[pl.ds(r, CHUNK)], rows_v.at[0])
        pltpu.sync_copy(rows_v.at[0], outp_hbm.at[c, pl.ds(r, CHUNK)])


@functools.partial(
    pl.kernel,
    out_type=jax.ShapeDtypeStruct((NC, NP, D), _f32),
    mesh=_sc_mesh,
    scratch_types=[
        pltpu.VMEM((16, CHUNK), jnp.int32),
        pltpu.VMEM((CHUNK, D), _f32),
        pltpu.SemaphoreType.DMA,
        pltpu.VMEM_SHARED((NP, D), _f32),
    ],
)
def _sc_count(dst_hbm, z128_hbm, ones_hbm, outc_hbm, dst_v, rows_v, sem,
              acc_sh):
    c = lax.axis_index("c")
    s = lax.axis_index("s")
    wid = s * NC + c

    pltpu.sync_copy(z128_hbm, rows_v)
    for k in range(RPT // CHUNK):
        pltpu.sync_copy(rows_v, acc_sh.at[pl.ds(s * RPT + k * CHUNK, CHUNK)])
    plsc.subcore_barrier()

    pltpu.sync_copy(ones_hbm, rows_v)

    for blk in range(CPW // 16):
        pltpu.sync_copy(dst_hbm.at[pl.ds(wid * CPW + blk * 16, 16)], dst_v)
        descs = [pltpu.async_copy(rows_v, acc_sh.at[dst_v.at[k]], sem,
                                  add=True)
                 for k in range(16)]
        for d_ in descs:
            d_.wait()
    plsc.subcore_barrier()

    for k in range(RPT // CHUNK):
        r = s * RPT + k * CHUNK
        pltpu.sync_copy(acc_sh.at[pl.ds(r, CHUNK)], rows_v)
        pltpu.sync_copy(rows_v, outc_hbm.at[c, pl.ds(r, CHUNK)])


_BLK = 256
_NBLK = NP // _BLK


def _make_update(relu):
    def body(p_ref, inv_ref, h_ref, wl_ref, bl_ref, wr_ref, o_ref):
        mean = (p_ref[0] + p_ref[1]) * inv_ref[...]
        out = lax.dot_general(mean, wl_ref[...], (((1,), (1,)), ((), ())),
                              preferred_element_type=_f32)
        out += lax.dot_general(h_ref[...], wr_ref[...], (((1,), (1,)), ((), ())),
                               preferred_element_type=_f32)
        out += bl_ref[...]
        o_ref[...] = jnp.maximum(out, 0.0) if relu else out

    return pl.pallas_call(
        body,
        grid=(_NBLK,),
        in_specs=[
            pl.BlockSpec((2, _BLK, D), lambda i: (0, i, 0)),
            pl.BlockSpec((_BLK, 1), lambda i: (i, 0)),
            pl.BlockSpec((_BLK, D), lambda i: (i, 0)),
            pl.BlockSpec((D, D), lambda i: (0, 0)),
            pl.BlockSpec((1, D), lambda i: (0, 0)),
            pl.BlockSpec((D, D), lambda i: (0, 0)),
        ],
        out_specs=pl.BlockSpec((_BLK, D), lambda i: (i, 0)),
        out_shape=jax.ShapeDtypeStruct((NP, D), _f32),
    )


_update_relu = _make_update(True)


def _pool_body(batch_ref, p_ref, inv_ref, h2_ref, wl_ref, bl_ref, wr_ref,
               o_ref, a_acc, b_acc, c_acc):
    i = pl.program_id(0)

    @pl.when(i == 0)
    def _init():
        a_acc[...] = jnp.zeros((G, D), _f32)
        b_acc[...] = jnp.zeros((G, D), _f32)
        c_acc[...] = jnp.zeros((G, D), _f32)

    m3 = (p_ref[0] + p_ref[1]) * inv_ref[...]
    b = batch_ref[0]
    sel = (lax.broadcasted_iota(jnp.int32, (G, _BLK), 0) == b).astype(_f32)
    a_acc[...] += lax.dot_general(sel, m3, (((1,), (0,)), ((), ())),
                                  preferred_element_type=_f32)
    b_acc[...] += lax.dot_general(sel, h2_ref[...], (((1,), (0,)), ((), ())),
                                  preferred_element_type=_f32)
    c_acc[...] = c_acc[...] + jnp.sum(sel, axis=1, keepdims=True)

    @pl.when(i == _NBLK - 1)
    def _fin():
        cg = jnp.maximum(c_acc[...], 1.0)
        out = lax.dot_general(a_acc[...] / cg, wl_ref[...],
                              (((1,), (1,)), ((), ())),
                              preferred_element_type=_f32)
        out += lax.dot_general(b_acc[...] / cg, wr_ref[...],
                               (((1,), (1,)), ((), ())),
                               preferred_element_type=_f32)
        o_ref[...] = out + bl_ref[...]


_pool = pl.pallas_call(
    _pool_body,
    grid=(_NBLK,),
    in_specs=[
        pl.BlockSpec((1, 1, _BLK), lambda i: (i, 0, 0)),
        pl.BlockSpec((2, _BLK, D), lambda i: (0, i, 0)),
        pl.BlockSpec((_BLK, 1), lambda i: (i, 0)),
        pl.BlockSpec((_BLK, D), lambda i: (i, 0)),
        pl.BlockSpec((D, D), lambda i: (0, 0)),
        pl.BlockSpec((1, D), lambda i: (0, 0)),
        pl.BlockSpec((D, D), lambda i: (0, 0)),
    ],
    out_specs=pl.BlockSpec((G, D), lambda i: (0, 0)),
    out_shape=jax.ShapeDtypeStruct((G, D), _f32),
    scratch_shapes=[
        pltpu.VMEM((G, D), _f32),
        pltpu.VMEM((G, D), _f32),
        pltpu.VMEM((G, D), _f32),
    ],
)


def kernel(x, edge_index, batch, Wl1, bl1, Wr1, Wl2, bl2, Wr2, Wl3, bl3, Wr3):
    xp = jnp.zeros((NP, D), _f32).at[:N].set(x)
    src = jnp.concatenate(
        [edge_index[0], jnp.zeros((EP - E,), jnp.int32)]).reshape(-1, CHUNK)
    dst = jnp.concatenate(
        [edge_index[1], jnp.full((EP - E,), N, jnp.int32)]).reshape(-1, CHUNK)
    batchp = jnp.concatenate(
        [batch, jnp.full((NP - N,), 1 << 20, jnp.int32)]).reshape(_NBLK, 1, _BLK)
    z128 = jnp.zeros((CHUNK, D), _f32)
    ones128 = jnp.ones((CHUNK, D), _f32)

    cp = _sc_count(dst, z128, ones128)
    inv = (1.0 / jnp.clip(cp[0, :, 0] + cp[1, :, 0], 1.0, None)).reshape(NP, 1)

    p1 = _sc_agg(xp, src, dst, z128)
    h1 = _update_relu(p1, inv, xp, Wl1, bl1.reshape(1, D), Wr1)
    p2 = _sc_agg(h1, src, dst, z128)
    h2 = _update_relu(p2, inv, h1, Wl2, bl2.reshape(1, D), Wr2)
    p3 = _sc_agg(h2, src, dst, z128)
    return _pool(batchp, p3, inv, h2, Wl3, bl3.reshape(1, D), Wr3)


# speedup vs baseline: 9.7247x; 2.8278x over previous
import functools

import jax
import jax.numpy as jnp
from jax import lax
from jax.experimental import pallas as pl
from jax.experimental.pallas import tpu as pltpu
from jax.experimental.pallas import tpu_sc as plsc

N = 10000
E = 320000
D = 128
G = 64

NC = 2
NS = 16
NW = NC * NS

CHUNK = 128
EP = 327680
CPW = EP // (NW * CHUNK)
NP = 10240
RPT = NP // NS

_f32 = jnp.float32


_sc_mesh = plsc.VectorSubcoreMesh(
    core_axis_name="c", subcore_axis_name="s", num_cores=NC, num_subcores=NS
)


@functools.partial(
    pl.kernel,
    out_type=jax.ShapeDtypeStruct((NC, NP, D), _f32),
    mesh=_sc_mesh,
    scratch_types=[
        pltpu.VMEM((16, CHUNK), jnp.int32),
        pltpu.VMEM((16, CHUNK), jnp.int32),
        pltpu.VMEM((2, CHUNK, D), _f32),
        pltpu.SemaphoreType.DMA,
        pltpu.SemaphoreType.DMA,
        pltpu.VMEM_SHARED((NP, D), _f32),
    ],
)
def _sc_agg(h_hbm, src_hbm, dst_hbm, z128_hbm, outp_hbm,
            src_v, dst_v, rows_v, sem0, sem1, acc_sh):
    c = lax.axis_index("c")
    s = lax.axis_index("s")
    wid = s * NC + c
    sems = (sem0, sem1)

    pltpu.sync_copy(z128_hbm, rows_v.at[0])
    for k in range(RPT // CHUNK):
        pltpu.sync_copy(rows_v.at[0],
                        acc_sh.at[pl.ds(s * RPT + k * CHUNK, CHUNK)])
    plsc.subcore_barrier()

    for blk in range(CPW // 16):
        pltpu.sync_copy(src_hbm.at[pl.ds(wid * CPW + blk * 16, 16)], src_v)
        pltpu.sync_copy(dst_hbm.at[pl.ds(wid * CPW + blk * 16, 16)], dst_v)
        pend = pltpu.async_copy(h_hbm.at[src_v.at[0]], rows_v.at[0], sem0)
        for k in range(16):
            nxt = None
            if k + 1 < 16:
                nxt = pltpu.async_copy(h_hbm.at[src_v.at[k + 1]],
                                       rows_v.at[(k + 1) % 2],
                                       sems[(k + 1) % 2])
            pend.wait()
            pltpu.sync_copy(rows_v.at[k % 2], acc_sh.at[dst_v.at[k]], add=True)
            pend = nxt
    plsc.subcore_barrier()

    for k in range(RPT // CHUNK):
        r = s * RPT + k * CHUNK
        pltpu.sync_copy(acc_sh.at[pl.ds(r, CHUNK)], rows_v.at[0])
        pltpu.sync_copy(rows_v.at[0], outp_hbm.at[c, pl.ds(r, CHUNK)])


@functools.partial(
    pl.kernel,
    out_type=jax.ShapeDtypeStruct((NC, NP, D), _f32),
    mesh=_sc_mesh,
    scratch_types=[
        pltpu.VMEM((16, CHUNK), jnp.int32),
        pltpu.VMEM((CHUNK, D), _f32),
        pltpu.SemaphoreType.DMA,
        pltpu.VMEM_SHARED((NP, D), _f32),
    ],
)
def _sc_count(dst_hbm, z128_hbm, ones_hbm, outc_hbm, dst_v, rows_v, sem,
              acc_sh):
    c = lax.axis_index("c")
    s = lax.axis_index("s")
    wid = s * NC + c

    pltpu.sync_copy(z128_hbm, rows_v)
    for k in range(RPT // CHUNK):
        pltpu.sync_copy(rows_v, acc_sh.at[pl.ds(s * RPT + k * CHUNK, CHUNK)])
    plsc.subcore_barrier()

    pltpu.sync_copy(ones_hbm, rows_v)

    for blk in range(CPW // 16):
        pltpu.sync_copy(dst_hbm.at[pl.ds(wid * CPW + blk * 16, 16)], dst_v)
        descs = [pltpu.async_copy(rows_v, acc_sh.at[dst_v.at[k]], sem,
                                  add=True)
                 for k in range(16)]
        for d_ in descs:
            d_.wait()
    plsc.subcore_barrier()

    for k in range(RPT // CHUNK):
        r = s * RPT + k * CHUNK
        pltpu.sync_copy(acc_sh.at[pl.ds(r, CHUNK)], rows_v)
        pltpu.sync_copy(rows_v, outc_hbm.at[c, pl.ds(r, CHUNK)])


_BLK = 256
_NBLK = NP // _BLK


def _make_update(relu):
    def body(p_ref, inv_ref, h_ref, wl_ref, bl_ref, wr_ref, o_ref):
        mean = (p_ref[0] + p_ref[1]) * inv_ref[...]
        out = lax.dot_general(mean, wl_ref[...], (((1,), (1,)), ((), ())),
                              preferred_element_type=_f32)
        out += lax.dot_general(h_ref[...], wr_ref[...], (((1,), (1,)), ((), ())),
                               preferred_element_type=_f32)
        out += bl_ref[...]
        o_ref[...] = jnp.maximum(out, 0.0) if relu else out

    return pl.pallas_call(
        body,
        grid=(_NBLK,),
        in_specs=[
            pl.BlockSpec((2, _BLK, D), lambda i: (0, i, 0)),
            pl.BlockSpec((_BLK, 1), lambda i: (i, 0)),
            pl.BlockSpec((_BLK, D), lambda i: (i, 0)),
            pl.BlockSpec((D, D), lambda i: (0, 0)),
            pl.BlockSpec((1, D), lambda i: (0, 0)),
            pl.BlockSpec((D, D), lambda i: (0, 0)),
        ],
        out_specs=pl.BlockSpec((_BLK, D), lambda i: (i, 0)),
        out_shape=jax.ShapeDtypeStruct((NP, D), _f32),
    )


_update_relu = _make_update(True)


def _pool_body(batch_ref, p_ref, inv_ref, h2_ref, wl_ref, bl_ref, wr_ref,
               o_ref, a_acc, b_acc, c_acc):
    i = pl.program_id(0)

    @pl.when(i == 0)
    def _init():
        a_acc[...] = jnp.zeros((G, D), _f32)
        b_acc[...] = jnp.zeros((G, D), _f32)
        c_acc[...] = jnp.zeros((G, D), _f32)

    m3 = (p_ref[0] + p_ref[1]) * inv_ref[...]
    b = batch_ref[0]
    sel = (lax.broadcasted_iota(jnp.int32, (G, _BLK), 0) == b).astype(_f32)
    a_acc[...] += lax.dot_general(sel, m3, (((1,), (0,)), ((), ())),
                                  preferred_element_type=_f32)
    b_acc[...] += lax.dot_general(sel, h2_ref[...], (((1,), (0,)), ((), ())),
                                  preferred_element_type=_f32)
    c_acc[...] = c_acc[...] + jnp.sum(sel, axis=1, keepdims=True)

    @pl.when(i == _NBLK - 1)
    def _fin():
        cg = jnp.maximum(c_acc[...], 1.0)
        out = lax.dot_general(a_acc[...] / cg, wl_ref[...],
                              (((1,), (1,)), ((), ())),
                              preferred_element_type=_f32)
        out += lax.dot_general(b_acc[...] / cg, wr_ref[...],
                               (((1,), (1,)), ((), ())),
                               preferred_element_type=_f32)
        o_ref[...] = out + bl_ref[...]


_pool = pl.pallas_call(
    _pool_body,
    grid=(_NBLK,),
    in_specs=[
        pl.BlockSpec((1, 1, _BLK), lambda i: (i, 0, 0)),
        pl.BlockSpec((2, _BLK, D), lambda i: (0, i, 0)),
        pl.BlockSpec((_BLK, 1), lambda i: (i, 0)),
        pl.BlockSpec((_BLK, D), lambda i: (i, 0)),
        pl.BlockSpec((D, D), lambda i: (0, 0)),
        pl.BlockSpec((1, D), lambda i: (0, 0)),
        pl.BlockSpec((D, D), lambda i: (0, 0)),
    ],
    out_specs=pl.BlockSpec((G, D), lambda i: (0, 0)),
    out_shape=jax.ShapeDtypeStruct((G, D), _f32),
    scratch_shapes=[
        pltpu.VMEM((G, D), _f32),
        pltpu.VMEM((G, D), _f32),
        pltpu.VMEM((G, D), _f32),
    ],
)


def kernel(x, edge_index, batch, Wl1, bl1, Wr1, Wl2, bl2, Wr2, Wl3, bl3, Wr3):
    xp = jnp.zeros((NP, D), _f32).at[:N].set(x)
    pad_ix = N + (jnp.arange(EP - E, dtype=jnp.int32) % (NP - N))
    src = jnp.concatenate([edge_index[0], pad_ix]).reshape(-1, CHUNK)
    dst = jnp.concatenate([edge_index[1], pad_ix]).reshape(-1, CHUNK)
    batchp = jnp.concatenate(
        [batch, jnp.full((NP - N,), 1 << 20, jnp.int32)]).reshape(_NBLK, 1, _BLK)
    z128 = jnp.zeros((CHUNK, D), _f32)
    ones128 = jnp.ones((CHUNK, D), _f32)

    cp = _sc_count(dst, z128, ones128)
    inv = (1.0 / jnp.clip(cp[0, :, 0] + cp[1, :, 0], 1.0, None)).reshape(NP, 1)

    p1 = _sc_agg(xp, src, dst, z128)
    h1 = _update_relu(p1, inv, xp, Wl1, bl1.reshape(1, D), Wr1)
    p2 = _sc_agg(h1, src, dst, z128)
    h2 = _update_relu(p2, inv, h1, Wl2, bl2.reshape(1, D), Wr2)
    p3 = _sc_agg(h2, src, dst, z128)
    return _pool(batchp, p3, inv, h2, Wl3, bl3.reshape(1, D), Wr3)
